# baseline (speedup 1.0000x reference)
import functools

import jax
import jax.numpy as jnp
from jax import lax
from jax.experimental import pallas as pl
from jax.experimental.pallas import tpu as pltpu
from jax.experimental.pallas import tpu_sc as plsc

VOCAB = 1000
D = 1000
B, T = 4096, 20
N_IDX = B * T

_info = plsc.get_sparse_core_info()
NC, NS = _info.num_cores, _info.num_subcores
NW = NC * NS
B_PER_W = B // NW
I_PER_W = B_PER_W * T
CB = 2
CHUNK = CB * T
NBUF = 2
NCHUNK = B_PER_W // CB
NROUND = NCHUNK // NBUF

_mesh = plsc.VectorSubcoreMesh(core_axis_name="c", subcore_axis_name="s")


@functools.partial(
    pl.kernel,
    mesh=_mesh,
    out_type=jax.ShapeDtypeStruct((B, T, D), jnp.float32),
    scratch_types=[
        pltpu.VMEM((I_PER_W,), jnp.int32),
        pltpu.VMEM((NBUF, CHUNK, D), jnp.float32),
        [pltpu.SemaphoreType.DMA] * NBUF,
        [pltpu.SemaphoreType.DMA] * NBUF,
    ],
    compiler_params=pltpu.CompilerParams(use_tc_tiling_on_sc=False),
)
def _gather_kernel(idx_hbm, table_hbm, out_hbm, idx_v, rows_v, gsem, wsem):
    wid = lax.axis_index("s") * NC + lax.axis_index("c")
    ibase = wid * I_PER_W
    bbase = wid * B_PER_W

    def g_copy(c, b):
        return pltpu.make_async_copy(
            table_hbm.at[idx_v.at[pl.ds(c * CHUNK, CHUNK)]], rows_v.at[b], gsem[b])

    def w_copy(c, b, j):
        return pltpu.make_async_copy(
            rows_v.at[b].at[pl.ds(j * T, T)],
            out_hbm.at[bbase + c * CB + j], wsem[b])

    pltpu.sync_copy(idx_hbm.at[pl.ds(ibase, I_PER_W)], idx_v)

    for b in range(NBUF):
        g_copy(b, b).start()

    def round_body(r, carry):
        for b in range(NBUF):
            c = r * NBUF + b
            g_copy(c, b).wait()
            for j in range(CB):
                w_copy(c, b, j).start()
            for j in range(CB):
                w_copy(c, b, j).wait()
            g_copy(c + NBUF, b).start()
        return carry

    lax.fori_loop(0, NROUND - 1, round_body, 0)

    last = (NROUND - 1) * NBUF
    for b in range(NBUF):
        g_copy(last + b, b).wait()
        for j in range(CB):
            w_copy(last + b, b, j).start()
    for b in range(NBUF):
        for j in range(CB):
            w_copy(last + b, b, j).wait()


def kernel(index, table):
    idx_flat = index.reshape(-1).astype(jnp.int32)
    return _gather_kernel(idx_flat, table)


# speedup vs baseline: 1.3514x; 1.3514x over previous
import functools

import jax
import jax.numpy as jnp
from jax import lax
from jax.experimental import pallas as pl
from jax.experimental.pallas import tpu as pltpu
from jax.experimental.pallas import tpu_sc as plsc

VOCAB = 1000
D = 1000
SEG = 128
NMAIN = 7
B, T = 4096, 20
TP = 24

_info = plsc.get_sparse_core_info()
NC, NS = _info.num_cores, _info.num_subcores
NW = NC * NS
B_PER_W = B // NW
NBUF = 4
NROUND = B_PER_W // NBUF

_mesh = plsc.VectorSubcoreMesh(core_axis_name="c", subcore_axis_name="s")


@functools.partial(
    pl.kernel,
    mesh=_mesh,
    out_type=jax.ShapeDtypeStruct((B, T, D), jnp.float32),
    scratch_types=[
        pltpu.VMEM((B_PER_W * TP,), jnp.int32),
        [pltpu.VMEM((T, D), jnp.float32) for _ in range(NBUF)],
        [pltpu.VMEM((T, SEG), jnp.float32) for _ in range(NBUF)],
        [pltpu.SemaphoreType.DMA] * NBUF,
        [pltpu.SemaphoreType.DMA] * NBUF,
    ],
    compiler_params=pltpu.CompilerParams(
        use_tc_tiling_on_sc=True, needs_layout_passes=False),
)
def _gather_kernel(idx_hbm, *rest):
    segs = rest[:NMAIN + 1]
    out_hbm = rest[NMAIN + 1]
    idx_v, rows, tails, gsem, wsem = rest[NMAIN + 2:]
    wid = lax.axis_index("s") * NC + lax.axis_index("c")
    bbase = wid * B_PER_W

    def g_copy(k, s, ct):
        idx = idx_v.at[pl.ds(k * TP, T)]
        if ct < NMAIN:
            dst = rows[s].at[:, pl.ds(ct * SEG, SEG)]
        else:
            dst = tails[s]
        return pltpu.make_async_copy(segs[ct].at[idx], dst, gsem[s])

    def w_copy(k, s):
        return pltpu.make_async_copy(rows[s], out_hbm.at[bbase + k], wsem[s])

    def tail_fix(s):
        lanes = lax.iota(jnp.int32, 16)
        idx_c = 992 + jnp.minimum(lanes, 7)
        mask = lanes < 8

        def per_t(t, carry):
            for j in range(6):
                rows[s][t, pl.ds(896 + 16 * j, 16)] = tails[s][t, pl.ds(16 * j, 16)]
            x = tails[s][t, pl.ds(96, 16)]
            idx_t = jnp.full((16,), t, jnp.int32)
            plsc.store_scatter(rows[s], [idx_t, idx_c], x, mask=mask)
            return carry
        lax.fori_loop(0, T, per_t, 0)

    pltpu.sync_copy(idx_hbm.at[pl.ds(bbase * TP, B_PER_W * TP)], idx_v)

    for s in range(NBUF):
        for ct in range(NMAIN + 1):
            g_copy(s, s, ct).start()

    def round_body(r, carry):
        for s in range(NBUF):
            k = r * NBUF + s
            for ct in range(NMAIN + 1):
                g_copy(k, s, ct).wait()
            tail_fix(s)
            w_copy(k, s).start()
            w_copy(k, s).wait()
            for ct in range(NMAIN + 1):
                g_copy(k + NBUF, s, ct).start()
        return carry

    lax.fori_loop(0, NROUND - 1, round_body, 0)

    last = (NROUND - 1) * NBUF
    for s in range(NBUF):
        for ct in range(NMAIN + 1):
            g_copy(last + s, s, ct).wait()
        tail_fix(s)
        w_copy(last + s, s).start()
    for s in range(NBUF):
        w_copy(last + s, s).wait()


def kernel(index, table):
    idx_pad = jnp.pad(index.astype(jnp.int32), ((0, 0), (0, TP - T)))
    segs = [table[:, ct * SEG:(ct + 1) * SEG] for ct in range(NMAIN)]
    segs.append(jnp.pad(table[:, NMAIN * SEG:], ((0, 0), (0, NMAIN * SEG + SEG - D))))
    return _gather_kernel(idx_pad.reshape(-1), *segs)
